# Initial kernel scaffold; baseline (speedup 1.0000x reference)
import jax
import jax.numpy as jnp
from jax.experimental import pallas as pl


def kernel(pred, target):
    raise NotImplementedError("write your pallas kernel here")


# speedup vs baseline: 10.1680x; 10.1680x over previous
import jax
import jax.numpy as jnp
from jax import lax
from jax.experimental import pallas as pl
from jax.experimental.pallas import tpu as pltpu

_CLASS_WEIGHT = (1.4543, 43.8739, 34.241, 47.3663, 27.4869)
_THRESH = 0.7
_MIN_KEPT = 256

_N, _C, _H, _W = 8, 5, 512, 512
_NPIX = _N * _H * _W
_ROWS, _COLS = 2048, 1024
_HBLK = 64
_CH = 128
_NCH = _ROWS // _CH
_RBLK = 64


def _stats_kernel(pred_ref, tgt_ref, score_ref, wn_ref, w_ref):
    p = pred_ref[0]
    t = tgt_ref[0]
    m = jnp.max(p, axis=0)
    e = jnp.exp(p - m[None, :, :])
    s = jnp.sum(e, axis=0)
    score = 1.0 / s
    lse = m + jnp.log(s)
    p_t = jnp.zeros_like(m)
    w = jnp.zeros_like(m)
    for c in range(_C):
        sel = t == c
        p_t = jnp.where(sel, p[c], p_t)
        w = jnp.where(sel, jnp.float32(_CLASS_WEIGHT[c]), w)
    score_ref[0] = score
    wn_ref[0] = w * (lse - p_t)
    w_ref[0] = w


def _count_le(score_ref, bound):
    def body(k, acc):
        s = score_ref[pl.ds(k * _CH, _CH), :]
        u = lax.bitcast_convert_type(s, jnp.int32)
        return acc + jnp.sum((u <= bound).astype(jnp.int32))
    return lax.fori_loop(0, _NCH, body, jnp.int32(0))


def _select_kernel(score_ref, thr_ref):
    lo0 = lax.bitcast_convert_type(jnp.float32(0.1), jnp.int32)
    hi0 = lax.bitcast_convert_type(jnp.float32(1.0), jnp.int32)

    def vbody(_, lh):
        lo, hi = lh
        mid = lo + (hi - lo) // 2
        c = _count_le(score_ref, mid)
        take = c >= _MIN_KEPT
        return jnp.where(take, lo, mid + 1), jnp.where(take, mid, hi)

    vstar, _ = lax.fori_loop(0, 26, vbody, (lo0, hi0))
    c_lt = _count_le(score_ref, vstar - 1)
    mrank = _MIN_KEPT - c_lt

    def _count_eq_upto(bound):
        def body(k, acc):
            s = score_ref[pl.ds(k * _CH, _CH), :]
            u = lax.bitcast_convert_type(s, jnp.int32)
            row = lax.broadcasted_iota(jnp.int32, (_CH, _COLS), 0) + k * _CH
            col = lax.broadcasted_iota(jnp.int32, (_CH, _COLS), 1)
            idx = row * _COLS + col
            return acc + jnp.sum(((u == vstar) & (idx <= bound)).astype(jnp.int32))
        return lax.fori_loop(0, _NCH, body, jnp.int32(0))

    def ibody(_, lh):
        lo, hi = lh
        mid = lo + (hi - lo) // 2
        c = _count_eq_upto(mid)
        take = c >= mrank
        return jnp.where(take, lo, mid + 1), jnp.where(take, mid, hi)

    idx256, _ = lax.fori_loop(0, 21, ibody, (jnp.int32(0), jnp.int32(_NPIX - 1)))
    tidx = jnp.maximum(jnp.int32(_MIN_KEPT), idx256)

    def gbody(k, acc):
        s = score_ref[pl.ds(k * _CH, _CH), :]
        row = lax.broadcasted_iota(jnp.int32, (_CH, _COLS), 0) + k * _CH
        col = lax.broadcasted_iota(jnp.int32, (_CH, _COLS), 1)
        idx = row * _COLS + col
        return acc + jnp.sum(jnp.where(idx == tidx, s, 0.0))
    threshold = lax.fori_loop(0, _NCH, gbody, jnp.float32(0.0))

    thr_ref[0, 0] = jnp.maximum(threshold, jnp.float32(_THRESH))


def _reduce_kernel(thr_ref, score_ref, wn_ref, w_ref, out_ref, acc_ref):
    i = pl.program_id(0)

    @pl.when(i == 0)
    def _():
        acc_ref[0] = 0.0
        acc_ref[1] = 0.0

    thr = thr_ref[0, 0]
    keep = score_ref[...] <= thr
    acc_ref[0] += jnp.sum(jnp.where(keep, wn_ref[...], 0.0))
    acc_ref[1] += jnp.sum(jnp.where(keep, w_ref[...], 0.0))

    @pl.when(i == pl.num_programs(0) - 1)
    def _():
        out_ref[0, 0] = acc_ref[0] / acc_ref[1]


def kernel(pred, target):
    nblk = _H // _HBLK
    flat = jax.ShapeDtypeStruct((_N, _H, _W), jnp.float32)
    score, wn, w = pl.pallas_call(
        _stats_kernel,
        grid=(_N, nblk),
        in_specs=[
            pl.BlockSpec((1, _C, _HBLK, _W), lambda i, j: (i, 0, j, 0)),
            pl.BlockSpec((1, _HBLK, _W), lambda i, j: (i, j, 0)),
        ],
        out_specs=[
            pl.BlockSpec((1, _HBLK, _W), lambda i, j: (i, j, 0)),
            pl.BlockSpec((1, _HBLK, _W), lambda i, j: (i, j, 0)),
            pl.BlockSpec((1, _HBLK, _W), lambda i, j: (i, j, 0)),
        ],
        out_shape=[flat, flat, flat],
    )(pred, target)

    score = score.reshape(_ROWS, _COLS)
    wn = wn.reshape(_ROWS, _COLS)
    w = w.reshape(_ROWS, _COLS)

    thr = pl.pallas_call(
        _select_kernel,
        out_shape=jax.ShapeDtypeStruct((1, 1), jnp.float32),
        out_specs=pl.BlockSpec(memory_space=pltpu.SMEM),
    )(score)

    nr = _ROWS // _RBLK
    loss = pl.pallas_call(
        _reduce_kernel,
        grid=(nr,),
        in_specs=[
            pl.BlockSpec(memory_space=pltpu.SMEM),
            pl.BlockSpec((_RBLK, _COLS), lambda i: (i, 0)),
            pl.BlockSpec((_RBLK, _COLS), lambda i: (i, 0)),
            pl.BlockSpec((_RBLK, _COLS), lambda i: (i, 0)),
        ],
        out_specs=pl.BlockSpec(memory_space=pltpu.SMEM),
        out_shape=jax.ShapeDtypeStruct((1, 1), jnp.float32),
        scratch_shapes=[pltpu.SMEM((2,), jnp.float32)],
    )(thr, score, wn, w)

    return loss.reshape(())
